# baseline (speedup 1.0000x reference)
import functools

import jax
import jax.numpy as jnp
from jax import lax
from jax.experimental import pallas as pl
from jax.experimental.pallas import tpu as pltpu
from jax.experimental.pallas import tpu_sc as plsc

_D = 64
_TOK_V = 100000
_PHR_V = 8192
_SLOW_BASE = 100352
_COMB_ROWS = _SLOW_BASE + _PHR_V


_CBLK = 2048
_FAST_CBLOCKS = 49


def _normalize_body(fast_ref, slow_ref, out_ref):
    i = pl.program_id(0)

    def norm_cols(x):
        s = jnp.sum(x * x, axis=0, keepdims=True)
        return x * (1.0 / (jnp.sqrt(s) + 1e-8))

    @pl.when(i < _FAST_CBLOCKS)
    def _():
        out_ref[...] = norm_cols(fast_ref[...])

    @pl.when(i >= _FAST_CBLOCKS)
    def _():
        out_ref[...] = norm_cols(slow_ref[...])


def _normalize_tables(fast_t, slow_t):
    grid = _COMB_ROWS // _CBLK
    return pl.pallas_call(
        _normalize_body,
        grid=(grid,),
        in_specs=[
            pl.BlockSpec((_D, _CBLK),
                         lambda i: (0, jnp.minimum(i, _FAST_CBLOCKS - 1))),
            pl.BlockSpec((_D, _CBLK),
                         lambda i: (0, jnp.clip(i - _FAST_CBLOCKS, 0, 3))),
        ],
        out_specs=pl.BlockSpec((_D, _CBLK), lambda i: (0, i)),
        out_shape=jax.ShapeDtypeStruct((_D, _COMB_ROWS), jnp.float32),
    )(fast_t, slow_t)


def _make_gather(n_rows: int):
    info = plsc.get_sparse_core_info()
    nc, ns = info.num_cores, info.num_subcores
    nw = nc * ns
    per_w = n_rows // nw
    chunk = 128
    cpg = 5
    grp = chunk * cpg
    n_grp = per_w // grp

    mesh = plsc.VectorSubcoreMesh(core_axis_name="c", subcore_axis_name="s")

    @functools.partial(
        pl.kernel,
        mesh=mesh,
        out_type=jax.ShapeDtypeStruct((n_rows, _D), jnp.float32),
        compiler_params=pltpu.CompilerParams(use_tc_tiling_on_sc=False),
        scratch_types=[
            pltpu.VMEM((per_w,), jnp.int32),
            pltpu.VMEM((grp, _D), jnp.float32),
            pltpu.VMEM((grp, _D), jnp.float32),
            pltpu.SemaphoreType.DMA,
            pltpu.SemaphoreType.DMA,
            pltpu.SemaphoreType.DMA,
            pltpu.SemaphoreType.DMA,
        ],
    )
    def gather_k(table_hbm, idx_hbm, out_hbm, idx_v, buf_a, buf_b,
                 gsem_a, gsem_b, wsem_a, wsem_b):
        wid = lax.axis_index("s") * nc + lax.axis_index("c")
        base0 = wid * per_w
        pltpu.sync_copy(idx_hbm.at[pl.ds(base0, per_w)], idx_v)

        def run_group(t, g, buf, gsem, wsem):
            @pl.when(t > 0)
            def _():
                pltpu.make_async_copy(
                    buf, out_hbm.at[pl.ds(base0, grp)], wsem).wait()
            l0 = g * grp
            cps = [
                pltpu.async_copy(
                    table_hbm.at[idx_v.at[pl.ds(l0 + b * chunk, chunk)]],
                    buf.at[pl.ds(b * chunk, chunk)],
                    gsem,
                )
                for b in range(cpg)
            ]
            for cp in cps:
                cp.wait()
            pltpu.async_copy(buf, out_hbm.at[pl.ds(base0 + l0, grp)], wsem)

        def body(t, carry):
            run_group(t, 2 * t, buf_a, gsem_a, wsem_a)
            run_group(t, 2 * t + 1, buf_b, gsem_b, wsem_b)
            return carry

        lax.fori_loop(0, n_grp // 2, body, 0)
        pltpu.make_async_copy(buf_a, out_hbm.at[pl.ds(base0, grp)], wsem_a).wait()
        pltpu.make_async_copy(buf_b, out_hbm.at[pl.ds(base0, grp)], wsem_b).wait()

    return gather_k


def kernel(token_ids, phrase_ids, fast_table, slow_table):
    b, l = token_ids.shape
    n = b * l
    tok = token_ids.T.reshape(-1).astype(jnp.int32)
    phr = phrase_ids.T.reshape(-1).astype(jnp.int32) + jnp.int32(_SLOW_BASE)
    idx2 = jnp.stack([tok, phr], axis=-1).reshape(-1)

    comb = _normalize_tables(fast_table.T, slow_table.T).T
    out = _make_gather(2 * n)(comb, idx2)
    return out.reshape(l, b, 2 * _D).transpose(1, 0, 2)


# speedup vs baseline: 1.0875x; 1.0875x over previous
import functools

import jax
import jax.numpy as jnp
from jax import lax
from jax.experimental import pallas as pl
from jax.experimental.pallas import tpu as pltpu
from jax.experimental.pallas import tpu_sc as plsc

_D = 64
_TOK_V = 100000
_PHR_V = 8192
_SLOW_BASE = 100352
_COMB_ROWS = _SLOW_BASE + _PHR_V


_CBLK = 2048
_FAST_CBLOCKS = 49


def _normalize_body(fast_ref, slow_ref, out_ref):
    i = pl.program_id(0)

    def norm_cols(x):
        s = jnp.sum(x * x, axis=0, keepdims=True)
        y = x * (1.0 / (jnp.sqrt(s) + 1e-8))
        return y.T

    @pl.when(i < _FAST_CBLOCKS)
    def _():
        out_ref[...] = norm_cols(fast_ref[...])

    @pl.when(i >= _FAST_CBLOCKS)
    def _():
        out_ref[...] = norm_cols(slow_ref[...])


def _normalize_tables(fast_t, slow_t):
    grid = _COMB_ROWS // _CBLK
    return pl.pallas_call(
        _normalize_body,
        grid=(grid,),
        in_specs=[
            pl.BlockSpec((_D, _CBLK),
                         lambda i: (0, jnp.minimum(i, _FAST_CBLOCKS - 1))),
            pl.BlockSpec((_D, _CBLK),
                         lambda i: (0, jnp.clip(i - _FAST_CBLOCKS, 0, 3))),
        ],
        out_specs=pl.BlockSpec((_CBLK, _D), lambda i: (i, 0)),
        out_shape=jax.ShapeDtypeStruct((_COMB_ROWS, _D), jnp.float32),
    )(fast_t, slow_t)


def _make_gather(n_rows: int):
    info = plsc.get_sparse_core_info()
    nc, ns = info.num_cores, info.num_subcores
    nw = nc * ns
    per_w = n_rows // nw
    chunk = 128
    cpg = 5
    grp = chunk * cpg
    n_grp = per_w // grp

    mesh = plsc.VectorSubcoreMesh(core_axis_name="c", subcore_axis_name="s")

    @functools.partial(
        pl.kernel,
        mesh=mesh,
        out_type=jax.ShapeDtypeStruct((n_rows, _D), jnp.float32),
        compiler_params=pltpu.CompilerParams(use_tc_tiling_on_sc=False),
        scratch_types=[
            pltpu.VMEM((per_w,), jnp.int32),
            pltpu.VMEM((grp, _D), jnp.float32),
            pltpu.VMEM((grp, _D), jnp.float32),
            pltpu.SemaphoreType.DMA,
            pltpu.SemaphoreType.DMA,
            pltpu.SemaphoreType.DMA,
            pltpu.SemaphoreType.DMA,
        ],
    )
    def gather_k(table_hbm, idx_hbm, out_hbm, idx_v, buf_a, buf_b,
                 gsem_a, gsem_b, wsem_a, wsem_b):
        wid = lax.axis_index("s") * nc + lax.axis_index("c")
        base0 = wid * per_w
        pltpu.sync_copy(idx_hbm.at[pl.ds(base0, per_w)], idx_v)

        def run_group(t, g, buf, gsem, wsem):
            @pl.when(t > 0)
            def _():
                pltpu.make_async_copy(
                    buf, out_hbm.at[pl.ds(base0, grp)], wsem).wait()
            l0 = g * grp
            cps = [
                pltpu.async_copy(
                    table_hbm.at[idx_v.at[pl.ds(l0 + b * chunk, chunk)]],
                    buf.at[pl.ds(b * chunk, chunk)],
                    gsem,
                )
                for b in range(cpg)
            ]
            for cp in cps:
                cp.wait()
            pltpu.async_copy(buf, out_hbm.at[pl.ds(base0 + l0, grp)], wsem)

        def body(t, carry):
            run_group(t, 2 * t, buf_a, gsem_a, wsem_a)
            run_group(t, 2 * t + 1, buf_b, gsem_b, wsem_b)
            return carry

        lax.fori_loop(0, n_grp // 2, body, 0)
        pltpu.make_async_copy(buf_a, out_hbm.at[pl.ds(base0, grp)], wsem_a).wait()
        pltpu.make_async_copy(buf_b, out_hbm.at[pl.ds(base0, grp)], wsem_b).wait()

    return gather_k


def kernel(token_ids, phrase_ids, fast_table, slow_table):
    b, l = token_ids.shape
    n = b * l
    tok = token_ids.T.reshape(-1).astype(jnp.int32)
    phr = phrase_ids.T.reshape(-1).astype(jnp.int32) + jnp.int32(_SLOW_BASE)
    idx2 = jnp.stack([tok, phr], axis=-1).reshape(-1)

    comb = _normalize_tables(fast_table.T, slow_table.T)
    out = _make_gather(2 * n)(comb, idx2)
    return out.reshape(l, b, 2 * _D).transpose(1, 0, 2)


# speedup vs baseline: 1.8010x; 1.6561x over previous
import functools

import jax
import jax.numpy as jnp
from jax import lax
from jax.experimental import pallas as pl
from jax.experimental.pallas import tpu as pltpu
from jax.experimental.pallas import tpu_sc as plsc

_D = 64
_TOK_V = 100000
_PHR_V = 8192
_CBLK = 4096
_FAST_CBLOCKS = 25
_SLOW_BASE = _FAST_CBLOCKS * _CBLK
_COMB_ROWS = _SLOW_BASE + _PHR_V


def _normalize_body(fast_ref, slow_ref, out_ref):
    i = pl.program_id(0)

    def norm_cols(x):
        s = jnp.sum(x * x, axis=0, keepdims=True)
        y = x * (1.0 / (jnp.sqrt(s) + 1e-8))
        return y.T

    @pl.when(i < _FAST_CBLOCKS)
    def _():
        out_ref[...] = norm_cols(fast_ref[...])

    @pl.when(i >= _FAST_CBLOCKS)
    def _():
        out_ref[...] = norm_cols(slow_ref[...])


def _normalize_tables(fast_t, slow_t):
    grid = _COMB_ROWS // _CBLK
    return pl.pallas_call(
        _normalize_body,
        grid=(grid,),
        in_specs=[
            pl.BlockSpec((_D, _CBLK),
                         lambda i: (0, jnp.minimum(i, _FAST_CBLOCKS - 1))),
            pl.BlockSpec((_D, _CBLK),
                         lambda i: (0, jnp.clip(i - _FAST_CBLOCKS, 0, 1))),
        ],
        out_specs=pl.BlockSpec((_CBLK, _D), lambda i: (i, 0)),
        out_shape=jax.ShapeDtypeStruct((_COMB_ROWS, _D), jnp.float32),
    )(fast_t, slow_t)


def _make_gather(n_rows: int):
    info = plsc.get_sparse_core_info()
    nc, ns = info.num_cores, info.num_subcores
    nw = nc * ns
    chunk = 128
    rows_w = n_rows // (nw * chunk)
    cpg = 5
    grp = chunk * cpg
    n_grp = rows_w // cpg
    per_w = rows_w * chunk

    mesh = plsc.VectorSubcoreMesh(core_axis_name="c", subcore_axis_name="s")

    @functools.partial(
        pl.kernel,
        mesh=mesh,
        out_type=jax.ShapeDtypeStruct((n_rows, _D), jnp.float32),
        compiler_params=pltpu.CompilerParams(use_tc_tiling_on_sc=False),
        scratch_types=[
            pltpu.VMEM((rows_w, chunk), jnp.int32),
            pltpu.VMEM((grp, _D), jnp.float32),
            pltpu.VMEM((grp, _D), jnp.float32),
            pltpu.SemaphoreType.DMA,
            pltpu.SemaphoreType.DMA,
            pltpu.SemaphoreType.DMA,
            pltpu.SemaphoreType.DMA,
        ],
    )
    def gather_k(table_hbm, idx_hbm, out_hbm, idx_v, buf_a, buf_b,
                 gsem_a, gsem_b, wsem_a, wsem_b):
        wid = lax.axis_index("s") * nc + lax.axis_index("c")
        base0 = wid * per_w
        pltpu.sync_copy(idx_hbm.at[pl.ds(wid * rows_w, rows_w)], idx_v)

        def run_group(t, g, buf, gsem, wsem):
            @pl.when(t > 0)
            def _():
                pltpu.make_async_copy(
                    buf, out_hbm.at[pl.ds(base0, grp)], wsem).wait()
            cps = [
                pltpu.async_copy(
                    table_hbm.at[idx_v.at[g * cpg + b]],
                    buf.at[pl.ds(b * chunk, chunk)],
                    gsem,
                )
                for b in range(cpg)
            ]
            for cp in cps:
                cp.wait()
            pltpu.async_copy(
                buf, out_hbm.at[pl.ds(base0 + g * grp, grp)], wsem)

        def body(t, carry):
            run_group(t, 2 * t, buf_a, gsem_a, wsem_a)
            run_group(t, 2 * t + 1, buf_b, gsem_b, wsem_b)
            return carry

        lax.fori_loop(0, n_grp // 2, body, 0)
        pltpu.make_async_copy(buf_a, out_hbm.at[pl.ds(base0, grp)], wsem_a).wait()
        pltpu.make_async_copy(buf_b, out_hbm.at[pl.ds(base0, grp)], wsem_b).wait()

    return gather_k


def kernel(token_ids, phrase_ids, fast_table, slow_table):
    b, l = token_ids.shape
    n = b * l
    tok3 = token_ids.T.reshape(2 * n // 128, 64).astype(jnp.int32)
    phr3 = phrase_ids.T.reshape(2 * n // 128, 64).astype(jnp.int32)
    lane = jax.lax.broadcasted_iota(jnp.int32, (2 * n // 128, 128), 1)
    idx2 = jnp.where(
        lane % 2 == 0,
        jnp.repeat(tok3, 2, axis=1),
        jnp.repeat(phr3 + jnp.int32(_SLOW_BASE), 2, axis=1),
    )

    comb = _normalize_tables(fast_table.T, slow_table.T)
    out = _make_gather(2 * n)(comb, idx2)
    return out.reshape(l, b, 2 * _D).transpose(1, 0, 2)


# speedup vs baseline: 2.2932x; 1.2733x over previous
import functools

import jax
import jax.numpy as jnp
from jax import lax
from jax.experimental import pallas as pl
from jax.experimental.pallas import tpu as pltpu
from jax.experimental.pallas import tpu_sc as plsc

_D = 64
_TOK_V = 100000
_PHR_V = 8192
_CBLK = 4096
_FAST_CBLOCKS = 25
_SLOW_BASE = _FAST_CBLOCKS * _CBLK
_COMB_ROWS = _SLOW_BASE + _PHR_V


def _normalize_body(fast_ref, slow_ref, out_ref):
    i = pl.program_id(0)

    def norm_cols(x):
        s = jnp.sum(x * x, axis=0, keepdims=True)
        y = x * (1.0 / (jnp.sqrt(s) + 1e-8))
        return jnp.concatenate(
            [y[:, : _CBLK // 2].T, y[:, _CBLK // 2 :].T], axis=-1)

    @pl.when(i < _FAST_CBLOCKS)
    def _():
        out_ref[...] = norm_cols(fast_ref[...])

    @pl.when(i >= _FAST_CBLOCKS)
    def _():
        out_ref[...] = norm_cols(slow_ref[...])


def _normalize_tables(fast_t, slow_t):
    grid = _COMB_ROWS // _CBLK
    return pl.pallas_call(
        _normalize_body,
        grid=(grid,),
        in_specs=[
            pl.BlockSpec((_D, _CBLK),
                         lambda i: (0, jnp.minimum(i, _FAST_CBLOCKS - 1))),
            pl.BlockSpec((_D, _CBLK),
                         lambda i: (0, jnp.clip(i - _FAST_CBLOCKS, 0, 1))),
        ],
        out_specs=pl.BlockSpec((_CBLK // 2, 2 * _D), lambda i: (i, 0)),
        out_shape=jax.ShapeDtypeStruct((_COMB_ROWS // 2, 2 * _D), jnp.float32),
    )(fast_t, slow_t)


def _make_gather(n_rows: int):
    info = plsc.get_sparse_core_info()
    nc, ns = info.num_cores, info.num_subcores
    nw = nc * ns
    chunk = 128
    rows_w = n_rows // (nw * chunk)
    cpg = 5
    grp = chunk * cpg
    n_grp = rows_w // cpg
    per_w = rows_w * chunk

    mesh = plsc.VectorSubcoreMesh(core_axis_name="c", subcore_axis_name="s")

    @functools.partial(
        pl.kernel,
        mesh=mesh,
        out_type=jax.ShapeDtypeStruct((n_rows, _D), jnp.float32),
        compiler_params=pltpu.CompilerParams(use_tc_tiling_on_sc=False),
        scratch_types=[
            pltpu.VMEM((rows_w, chunk), jnp.int32),
            pltpu.VMEM((grp, _D), jnp.float32),
            pltpu.VMEM((grp, _D), jnp.float32),
            pltpu.SemaphoreType.DMA,
            pltpu.SemaphoreType.DMA,
            pltpu.SemaphoreType.DMA,
            pltpu.SemaphoreType.DMA,
        ],
    )
    def gather_k(table_hbm, idx_hbm, out_hbm, idx_v, buf_a, buf_b,
                 gsem_a, gsem_b, wsem_a, wsem_b):
        wid = lax.axis_index("s") * nc + lax.axis_index("c")
        base0 = wid * per_w
        pltpu.sync_copy(idx_hbm.at[pl.ds(wid * rows_w, rows_w)], idx_v)

        def fire_group(t, g, buf, gsem, wsem):
            @pl.when(t > 0)
            def _():
                pltpu.make_async_copy(
                    buf, out_hbm.at[pl.ds(base0, grp)], wsem).wait()
            return [
                pltpu.async_copy(
                    table_hbm.at[idx_v.at[g * cpg + b]],
                    buf.at[pl.ds(b * chunk, chunk)],
                    gsem,
                )
                for b in range(cpg)
            ]

        def drain_group(g, buf, cps, wsem):
            for cp in cps:
                cp.wait()
            pltpu.async_copy(
                buf, out_hbm.at[pl.ds(base0 + g * grp, grp)], wsem)

        def body(t, carry):
            cps_a = fire_group(t, 2 * t, buf_a, gsem_a, wsem_a)
            cps_b = fire_group(t, 2 * t + 1, buf_b, gsem_b, wsem_b)
            drain_group(2 * t, buf_a, cps_a, wsem_a)
            drain_group(2 * t + 1, buf_b, cps_b, wsem_b)
            return carry

        lax.fori_loop(0, n_grp // 2, body, 0)
        pltpu.make_async_copy(buf_a, out_hbm.at[pl.ds(base0, grp)], wsem_a).wait()
        pltpu.make_async_copy(buf_b, out_hbm.at[pl.ds(base0, grp)], wsem_b).wait()

    return gather_k


def kernel(token_ids, phrase_ids, fast_table, slow_table):
    b, l = token_ids.shape
    n = b * l
    tok3 = token_ids.T.reshape(2 * n // 128, 64).astype(jnp.int32)
    phr3 = phrase_ids.T.reshape(2 * n // 128, 64).astype(jnp.int32)
    lane = jax.lax.broadcasted_iota(jnp.int32, (2 * n // 128, 128), 1)
    v = jnp.where(
        lane % 2 == 0,
        jnp.repeat(tok3, 2, axis=1),
        jnp.repeat(phr3 + jnp.int32(_SLOW_BASE), 2, axis=1),
    )
    half = _CBLK // 2
    idx2 = (v & ~(_CBLK - 1)) + (v & (half - 1)) * 2 + ((v // half) & 1)

    comb = _normalize_tables(fast_table.T, slow_table.T).reshape(_COMB_ROWS, _D)
    out = _make_gather(2 * n)(comb, idx2)
    return out.reshape(l, b, 2 * _D).transpose(1, 0, 2)


# speedup vs baseline: 2.3955x; 1.0446x over previous
import functools

import jax
import jax.numpy as jnp
from jax import lax
from jax.experimental import pallas as pl
from jax.experimental.pallas import tpu as pltpu
from jax.experimental.pallas import tpu_sc as plsc

_D = 64
_TOK_V = 100000
_PHR_V = 8192
_CBLK = 4096
_FAST_CBLOCKS = 25
_SLOW_BASE = _FAST_CBLOCKS * _CBLK
_COMB_ROWS = _SLOW_BASE + _PHR_V


def _normalize_body(fast_ref, slow_ref, out_ref):
    i = pl.program_id(0)

    def norm_cols(x):
        s = jnp.sum(x * x, axis=0, keepdims=True)
        y = x * (1.0 / (jnp.sqrt(s) + 1e-8))
        return jnp.concatenate(
            [y[:, : _CBLK // 2].T, y[:, _CBLK // 2 :].T], axis=-1)

    @pl.when(i < _FAST_CBLOCKS)
    def _():
        out_ref[...] = norm_cols(fast_ref[...])

    @pl.when(i >= _FAST_CBLOCKS)
    def _():
        out_ref[...] = norm_cols(slow_ref[...])


def _normalize_tables(fast_t, slow_t):
    grid = _COMB_ROWS // _CBLK
    return pl.pallas_call(
        _normalize_body,
        grid=(grid,),
        in_specs=[
            pl.BlockSpec((_D, _CBLK),
                         lambda i: (0, jnp.minimum(i, _FAST_CBLOCKS - 1))),
            pl.BlockSpec((_D, _CBLK),
                         lambda i: (0, jnp.clip(i - _FAST_CBLOCKS, 0, 1))),
        ],
        out_specs=pl.BlockSpec((_CBLK // 2, 2 * _D), lambda i: (i, 0)),
        out_shape=jax.ShapeDtypeStruct((_COMB_ROWS // 2, 2 * _D), jnp.float32),
    )(fast_t, slow_t)


def _make_gather(n_rows: int):
    info = plsc.get_sparse_core_info()
    nc, ns = info.num_cores, info.num_subcores
    nw = nc * ns
    chunk = 128
    rows_w = n_rows // (nw * chunk)
    slots = 10
    n_it = rows_w // slots
    per_w = rows_w * chunk

    mesh = plsc.VectorSubcoreMesh(core_axis_name="c", subcore_axis_name="s")

    @functools.partial(
        pl.kernel,
        mesh=mesh,
        out_type=jax.ShapeDtypeStruct((n_rows, _D), jnp.float32),
        compiler_params=pltpu.CompilerParams(use_tc_tiling_on_sc=False),
        scratch_types=[
            pltpu.VMEM((rows_w, chunk), jnp.int32),
            pltpu.VMEM((slots * chunk, _D), jnp.float32),
            pltpu.SemaphoreType.DMA((slots,)),
            pltpu.SemaphoreType.DMA((slots,)),
        ],
    )
    def gather_k(table_hbm, idx_hbm, out_hbm, idx_v, buf, gsem, wsem):
        wid = lax.axis_index("s") * nc + lax.axis_index("c")
        base0 = wid * per_w
        pltpu.sync_copy(idx_hbm.at[pl.ds(wid * rows_w, rows_w)], idx_v)

        def slot_buf(b):
            return buf.at[pl.ds(b * chunk, chunk)]

        def body(t, carry):
            cps = []
            for b in range(slots):
                @pl.when(t > 0)
                def _(b=b):
                    pltpu.make_async_copy(
                        slot_buf(b), out_hbm.at[pl.ds(base0, chunk)],
                        wsem.at[b]).wait()
                cps.append(pltpu.async_copy(
                    table_hbm.at[idx_v.at[t * slots + b]],
                    slot_buf(b), gsem.at[b]))
            for b in range(slots):
                cps[b].wait()
                pltpu.async_copy(
                    slot_buf(b),
                    out_hbm.at[pl.ds(base0 + (t * slots + b) * chunk, chunk)],
                    wsem.at[b])
            return carry

        lax.fori_loop(0, n_it, body, 0)
        for b in range(slots):
            pltpu.make_async_copy(
                slot_buf(b), out_hbm.at[pl.ds(base0, chunk)],
                wsem.at[b]).wait()

    return gather_k


def kernel(token_ids, phrase_ids, fast_table, slow_table):
    b, l = token_ids.shape
    n = b * l
    tok3 = token_ids.T.reshape(2 * n // 128, 64).astype(jnp.int32)
    phr3 = phrase_ids.T.reshape(2 * n // 128, 64).astype(jnp.int32)
    lane = jax.lax.broadcasted_iota(jnp.int32, (2 * n // 128, 128), 1)
    v = jnp.where(
        lane % 2 == 0,
        jnp.repeat(tok3, 2, axis=1),
        jnp.repeat(phr3 + jnp.int32(_SLOW_BASE), 2, axis=1),
    )
    half = _CBLK // 2
    idx2 = (v & ~(_CBLK - 1)) + (v & (half - 1)) * 2 + ((v // half) & 1)

    comb = _normalize_tables(fast_table.T, slow_table.T).reshape(_COMB_ROWS, _D)
    out = _make_gather(2 * n)(comb, idx2)
    return out.reshape(l, b, 2 * _D).transpose(1, 0, 2)


# speedup vs baseline: 2.4989x; 1.0432x over previous
import functools

import jax
import jax.numpy as jnp
from jax import lax
from jax.experimental import pallas as pl
from jax.experimental.pallas import tpu as pltpu
from jax.experimental.pallas import tpu_sc as plsc

_D = 64
_TOK_V = 100000
_PHR_V = 8192
_CBLK = 8192
_FAST_CBLOCKS = 13
_SLOW_BASE = _FAST_CBLOCKS * _CBLK
_COMB_ROWS = _SLOW_BASE + _PHR_V


def _normalize_body(fast_ref, slow_ref, out_ref):
    i = pl.program_id(0)

    def norm_cols(x):
        s = jnp.sum(x * x, axis=0, keepdims=True)
        y = x * (1.0 / (jnp.sqrt(s) + 1e-8))
        return jnp.concatenate(
            [y[:, : _CBLK // 2].T, y[:, _CBLK // 2 :].T], axis=-1)

    @pl.when(i < _FAST_CBLOCKS)
    def _():
        out_ref[...] = norm_cols(fast_ref[...])

    @pl.when(i >= _FAST_CBLOCKS)
    def _():
        out_ref[...] = norm_cols(slow_ref[...])


def _normalize_tables(fast_t, slow_t):
    grid = _COMB_ROWS // _CBLK
    return pl.pallas_call(
        _normalize_body,
        grid=(grid,),
        in_specs=[
            pl.BlockSpec((_D, _CBLK),
                         lambda i: (0, jnp.minimum(i, _FAST_CBLOCKS - 1))),
            pl.BlockSpec((_D, _CBLK),
                         lambda i: (0, jnp.clip(i - _FAST_CBLOCKS, 0, 0))),
        ],
        out_specs=pl.BlockSpec((_CBLK // 2, 2 * _D), lambda i: (i, 0)),
        out_shape=jax.ShapeDtypeStruct((_COMB_ROWS // 2, 2 * _D), jnp.float32),
    )(fast_t, slow_t)


def _make_gather(n_rows: int):
    info = plsc.get_sparse_core_info()
    nc, ns = info.num_cores, info.num_subcores
    nw = nc * ns
    chunk = 128
    rows_w = n_rows // (nw * chunk)
    slots = 10
    n_it = rows_w // slots
    per_w = rows_w * chunk

    mesh = plsc.VectorSubcoreMesh(core_axis_name="c", subcore_axis_name="s")

    @functools.partial(
        pl.kernel,
        mesh=mesh,
        out_type=jax.ShapeDtypeStruct((n_rows, _D), jnp.float32),
        compiler_params=pltpu.CompilerParams(use_tc_tiling_on_sc=False),
        scratch_types=[
            pltpu.VMEM((rows_w, chunk), jnp.int32),
            pltpu.VMEM((slots * chunk, _D), jnp.float32),
            pltpu.SemaphoreType.DMA((slots,)),
            pltpu.SemaphoreType.DMA((slots,)),
        ],
    )
    def gather_k(table_hbm, idx_hbm, out_hbm, idx_v, buf, gsem, wsem):
        wid = lax.axis_index("s") * nc + lax.axis_index("c")
        base0 = wid * per_w
        pltpu.sync_copy(idx_hbm.at[pl.ds(wid * rows_w, rows_w)], idx_v)

        def slot_buf(b):
            return buf.at[pl.ds(b * chunk, chunk)]

        def body(t, carry):
            cps = []
            for b in range(slots):
                @pl.when(t > 0)
                def _(b=b):
                    pltpu.make_async_copy(
                        slot_buf(b), out_hbm.at[pl.ds(base0, chunk)],
                        wsem.at[b]).wait()
                cps.append(pltpu.async_copy(
                    table_hbm.at[idx_v.at[t * slots + b]],
                    slot_buf(b), gsem.at[b]))
            for b in range(slots):
                cps[b].wait()
                pltpu.async_copy(
                    slot_buf(b),
                    out_hbm.at[pl.ds(base0 + (t * slots + b) * chunk, chunk)],
                    wsem.at[b])
            return carry

        lax.fori_loop(0, n_it, body, 0)
        for b in range(slots):
            pltpu.make_async_copy(
                slot_buf(b), out_hbm.at[pl.ds(base0, chunk)],
                wsem.at[b]).wait()

    return gather_k


def kernel(token_ids, phrase_ids, fast_table, slow_table):
    b, l = token_ids.shape
    n = b * l
    tok3 = token_ids.T.reshape(2 * n // 128, 64).astype(jnp.int32)
    phr3 = phrase_ids.T.reshape(2 * n // 128, 64).astype(jnp.int32)
    lane = jax.lax.broadcasted_iota(jnp.int32, (2 * n // 128, 128), 1)
    v = jnp.where(
        lane % 2 == 0,
        jnp.repeat(tok3, 2, axis=1),
        jnp.repeat(phr3 + jnp.int32(_SLOW_BASE), 2, axis=1),
    )
    half = _CBLK // 2
    idx2 = (v & ~(_CBLK - 1)) + (v & (half - 1)) * 2 + ((v // half) & 1)

    comb = _normalize_tables(fast_table.T, slow_table.T).reshape(_COMB_ROWS, _D)
    out = _make_gather(2 * n)(comb, idx2)
    return out.reshape(l, b, 2 * _D).transpose(1, 0, 2)
